# Initial kernel scaffold; baseline (speedup 1.0000x reference)
import jax
import jax.numpy as jnp
from jax.experimental import pallas as pl


def kernel(encoder_output, phone_durations_input, w1, b1, w2, b2, g1, be1, g2, be2, wd, bd, training):
    raise NotImplementedError("write your pallas kernel here")


# speedup vs baseline: 34.1281x; 34.1281x over previous
import functools

import jax
import jax.numpy as jnp
from jax import lax
from jax.experimental import pallas as pl
from jax.experimental.pallas import tpu as pltpu
from jax.experimental.pallas import tpu_sc as plsc

_B, _S, _D, _T = 8, 512, 256, 2048
_K = 9
_HALO = _K // 2
_L = 16
_NC, _NS = 2, 16
_NW = _NC * _NS
_TPR = _NW // _B
_SPAN = _T // _TPR
_CHUNK = 128
_NCHUNK = _SPAN // _CHUNK
_ZROWS = 128


def _tc_body(enc_ref, w1_ref, b1_ref, w2_ref, b2_ref, g1_ref, be1_ref,
             g2_ref, be2_ref, wd_ref, bd_ref, dur_ref):
    x = enc_ref[0]

    def conv_block(xin, w_ref, b_ref, g_ref, be_ref):
        xp = jnp.pad(xin, ((_HALO, _HALO), (0, 0)))
        acc = jnp.broadcast_to(b_ref[...], (_S, _D))
        for k in range(_K):
            acc = acc + jnp.dot(xp[k:k + _S, :], w_ref[k],
                                preferred_element_type=jnp.float32)
        y = jnp.maximum(acc, 0.0) + xin
        mu = jnp.mean(y, axis=1, keepdims=True)
        var = jnp.mean((y - mu) ** 2, axis=1, keepdims=True)
        return (y - mu) / jnp.sqrt(var + 1e-6) * g_ref[...] + be_ref[...]

    c1 = conv_block(x, w1_ref, b1_ref, g1_ref, be1_ref)
    c2 = conv_block(c1, w2_ref, b2_ref, g2_ref, be2_ref)
    r = jnp.dot(c2, wd_ref[...], preferred_element_type=jnp.float32)
    dur_ref[...] = (r + bd_ref[...])[None]


def _tc_conv(enc, w1, b1, w2, b2, g1, be1, g2, be2, wd, bd):
    vec = pl.BlockSpec((1, _D), lambda b: (0, 0))
    return pl.pallas_call(
        _tc_body,
        grid=(_B,),
        in_specs=[
            pl.BlockSpec((1, _S, _D), lambda b: (b, 0, 0)),
            pl.BlockSpec((_K, _D, _D), lambda b: (0, 0, 0)),
            vec,
            pl.BlockSpec((_K, _D, _D), lambda b: (0, 0, 0)),
            vec, vec, vec, vec, vec,
            pl.BlockSpec((_D, 1), lambda b: (0, 0)),
            pl.BlockSpec((1, 1), lambda b: (0, 0)),
        ],
        out_specs=pl.BlockSpec((1, _S, 1), lambda b: (b, 0, 0)),
        out_shape=jax.ShapeDtypeStruct((_B, _S, 1), jnp.float32),
    )(enc, w1, b1, w2, b2, g1, be1, g2, be2, wd, bd)


def _sc_regulate_body(enc_hbm, dur_hbm, out_hbm, mask_hbm,
                      dur_v, cnt_v, idx_v, mask_v, rows_v, zero_v, sem):
    cid = lax.axis_index("c")
    sid = lax.axis_index("s")
    wid = sid * _NC + cid
    b = wid // _TPR
    t0 = (wid % _TPR) * _SPAN
    row0 = b * _T + t0

    pltpu.sync_copy(dur_hbm, dur_v)

    zvec_i = jnp.zeros((_L,), jnp.int32)
    zvec_f = jnp.zeros((_L,), jnp.float32)

    def zero_cnt(j, c):
        cnt_v[pl.ds(j * _L, _L)] = zvec_i
        return c
    lax.fori_loop(0, _SPAN // _L, zero_cnt, 0)


    ones_i = jnp.ones((_L,), jnp.int32)

    def pass1(i, carry):
        total, base = carry
        d = jnp.minimum(jnp.maximum(dur_v[b, pl.ds(i * _L, _L)], 1.0), 6.0)
        n0 = d.astype(jnp.int32)
        fr = d - n0.astype(jnp.float32)
        up = (fr > 0.5) | ((fr == 0.5) & ((n0 & 1) == 1))
        di = n0 + up.astype(jnp.int32)
        cs = plsc.cumsum(di) + total
        rel = cs - t0
        m = (rel >= 0) & (rel < _SPAN)
        relc = jnp.minimum(jnp.maximum(rel, 0), _SPAN - 1)
        plsc.store_scatter(cnt_v, [relc], ones_i, mask=m)
        total = total + jnp.sum(di)
        base = base + jnp.sum((cs < t0).astype(jnp.int32))
        return total, base

    total, base = lax.fori_loop(0, _S // _L, pass1, (0, 0))

    def pass2(j, c):
        s = plsc.cumsum(cnt_v[pl.ds(j * _L, _L)]) + c
        idx = jnp.minimum(s, _S - 1)
        idx_v[pl.ds(j * _L, _L)] = idx
        t = t0 + j * _L + lax.iota(jnp.int32, _L)
        mask_v[pl.ds(j * _L, _L)] = jnp.where(t < total, 1.0, 0.0)
        return jnp.max(s)
    lax.fori_loop(0, _SPAN // _L, pass2, base)

    pltpu.sync_copy(mask_v, mask_hbm.at[pl.ds(b * _T + t0, _SPAN)])

    valid_n = jnp.minimum(jnp.maximum(total - t0, 0), _SPAN)

    @pl.when(valid_n < _SPAN)
    def _():
        def zero_rows(r, c):
            for k in range(_D // _L):
                zero_v[r, pl.ds(k * _L, _L)] = zvec_f
            return c
        lax.fori_loop(0, _ZROWS, zero_rows, 0)

    def chunk_body(c, carry):
        start = pl.multiple_of(c * _CHUNK, _CHUNK)
        nv = jnp.minimum(valid_n - start, _CHUNK)
        off = pl.multiple_of(row0 + start, _CHUNK)

        @pl.when(nv > 0)
        def _():
            pltpu.async_copy(
                enc_hbm.at[b].at[idx_v.at[pl.ds(start, _CHUNK)]], rows_v, sem
            ).wait()

            def ztail(r, cc):
                for k in range(_D // _L):
                    rows_v[r, pl.ds(k * _L, _L)] = zvec_f
                return cc
            lax.fori_loop(nv, _CHUNK, ztail, 0)
            pltpu.sync_copy(rows_v, out_hbm.at[pl.ds(off, _CHUNK)])

        @pl.when(nv <= 0)
        def _():
            pltpu.sync_copy(zero_v, out_hbm.at[pl.ds(off, _ZROWS)])
        return carry
    lax.fori_loop(0, _NCHUNK, chunk_body, 0)


def _sc_regulate(enc, dur):
    mesh = plsc.VectorSubcoreMesh(core_axis_name="c", subcore_axis_name="s",
                                  num_cores=_NC, num_subcores=_NS)
    kern = functools.partial(
        pl.kernel,
        out_type=(
            jax.ShapeDtypeStruct((_B * _T, _D), jnp.float32),
            jax.ShapeDtypeStruct((_B * _T,), jnp.float32),
        ),
        mesh=mesh,
        scratch_types=[
            pltpu.VMEM((_B, _S), jnp.float32),
            pltpu.VMEM((_SPAN,), jnp.int32),
            pltpu.VMEM((_SPAN,), jnp.int32),
            pltpu.VMEM((_SPAN,), jnp.float32),
            pltpu.VMEM((_CHUNK, _D), jnp.float32),
            pltpu.VMEM((_ZROWS, _D), jnp.float32),
            pltpu.SemaphoreType.DMA,
        ],
        compiler_params=pltpu.CompilerParams(needs_layout_passes=False),
    )(_sc_regulate_body)
    return kern(enc, dur)


def kernel(encoder_output, phone_durations_input, w1, b1, w2, b2,
           g1, be1, g2, be2, wd, bd, training=False):
    del training
    dur_col = _tc_conv(encoder_output, w1, b1.reshape(1, _D),
                       w2, b2.reshape(1, _D), g1.reshape(1, _D),
                       be1.reshape(1, _D), g2.reshape(1, _D),
                       be2.reshape(1, _D), wd, bd.reshape(1, 1))
    reg_flat, mask = _sc_regulate(encoder_output, phone_durations_input)
    return (reg_flat.reshape(_B, _T, _D),
            dur_col.reshape(_B, _S),
            mask.reshape(_B, _T))
